# baseline (speedup 1.0000x reference)
import functools

import jax
import jax.numpy as jnp
from jax import lax
from jax.experimental import pallas as pl
from jax.experimental.pallas import tpu as pltpu
from jax.experimental.pallas import tpu_sc as plsc

B, T, E, S, L, H = 8, 2048, 128, 512, 128, 150
NC, NS = 2, 16
NW = NC * NS
SPW = (B * S) // NW
IDW = SPW * L
HI = lax.Precision.DEFAULT


def _token_body(xt_ref, w1t, b1, w2, b2, w3t, b3, out_ref):
    xt = xt_ref[...]
    b1c = b1[...].reshape(H, 1)
    b2c = b2[...].reshape(H, 1)
    ht = jnp.maximum(jnp.dot(w1t[...], xt, precision=HI) + b1c, 0.0)
    ht = jnp.maximum(
        lax.dot_general(w2[...], ht, (((0,), (0,)), ((), ())),
                        precision=HI) + b2c, 0.0)
    sct = jnp.dot(w3t[...], ht, precision=HI) + b3[...]
    rst = jnp.sum(xt, axis=0, keepdims=True)
    out_ref[...] = (sct * rst).reshape(_TOKEN_BLK)


_TOKEN_BLK = 4096
_token_call = pl.pallas_call(
    _token_body,
    grid=((B * T) // _TOKEN_BLK,),
    in_specs=[
        pl.BlockSpec((E, _TOKEN_BLK), lambda i: (0, i)),
        pl.BlockSpec((H, E), lambda i: (0, 0)),
        pl.BlockSpec((H,), lambda i: (0,)),
        pl.BlockSpec((H, H), lambda i: (0, 0)),
        pl.BlockSpec((H,), lambda i: (0,)),
        pl.BlockSpec((1, H), lambda i: (0, 0)),
        pl.BlockSpec((1, 1), lambda i: (0, 0)),
    ],
    out_specs=pl.BlockSpec((_TOKEN_BLK,), lambda i: (i,)),
    out_shape=jax.ShapeDtypeStruct((B * T,), jnp.float32),
)


def _sc_body(embeds, ids, w, first_o, last_o, wgt_o,
             ids_v, w_v, fidx, lidx, rows_f, rows_l, out_w, sem):
    wid = lax.axis_index("s") * NC + lax.axis_index("c")
    b = wid // (NW // B)
    span0 = wid * SPW
    bT = b * T

    pltpu.sync_copy(ids.at[pl.ds(wid * IDW, IDW)], ids_v)

    iota = lax.iota(jnp.int32, 16)
    for j in range(SPW // 16):
        pos = (j * 16 + iota) * L
        fidx[pl.ds(j * 16, 16)] = plsc.load_gather(ids_v, [pos]) + bT
        lidx[pl.ds(j * 16, 16)] = plsc.load_gather(ids_v, [pos + (L - 1)]) + bT

    cf = pltpu.async_copy(embeds.at[fidx], rows_f, sem)
    cl = pltpu.async_copy(embeds.at[lidx], rows_l, sem)

    pltpu.sync_copy(w.at[pl.ds(bT, T)], w_v)

    @plsc.parallel_loop(0, IDW, 16, unroll=16)
    def _(off):
        idx = ids_v[pl.ds(off, 16)]
        out_w[pl.ds(off, 16)] = plsc.load_gather(w_v, [idx])

    cf.wait()
    cl.wait()
    pltpu.sync_copy(rows_f, first_o.at[pl.ds(span0, SPW)])
    pltpu.sync_copy(rows_l, last_o.at[pl.ds(span0, SPW)])
    pltpu.sync_copy(out_w, wgt_o.at[pl.ds(wid * IDW, IDW)])


_sc_call = pl.kernel(
    _sc_body,
    out_type=[
        jax.ShapeDtypeStruct((B * S, E), jnp.float32),
        jax.ShapeDtypeStruct((B * S, E), jnp.float32),
        jax.ShapeDtypeStruct((B * S * L,), jnp.float32),
    ],
    mesh=plsc.VectorSubcoreMesh(
        core_axis_name="c", subcore_axis_name="s",
        num_cores=NC, num_subcores=NS),
    compiler_params=pltpu.CompilerParams(needs_layout_passes=False),
    scratch_types=[
        pltpu.VMEM((IDW,), jnp.int32),
        pltpu.VMEM((T,), jnp.float32),
        pltpu.VMEM((SPW,), jnp.int32),
        pltpu.VMEM((SPW,), jnp.int32),
        pltpu.VMEM((SPW, E), jnp.float32),
        pltpu.VMEM((SPW, E), jnp.float32),
        pltpu.VMEM((IDW,), jnp.float32),
        pltpu.SemaphoreType.DMA,
    ],
)


def _span_body(f_ref, l_ref, g_ref, w1, b1, w2, b2, w3, b3, se_ref, ms_ref):
    f = f_ref[...]
    lst = l_ref[...]
    g = g_ref[...]
    se_ref[:, 0:E] = f
    se_ref[:, E:2 * E] = lst
    se_ref[:, 2 * E:3 * E] = g
    h = (jnp.dot(f, w1[0:E, :], precision=HI)
         + jnp.dot(lst, w1[E:2 * E, :], precision=HI)
         + jnp.dot(g, w1[2 * E:3 * E, :], precision=HI) + b1[...])
    h = jnp.maximum(h, 0.0)
    h = jnp.maximum(jnp.dot(h, w2[...], precision=HI) + b2[...], 0.0)
    ms = jnp.dot(h, w3[...], precision=HI) + b3[...]
    ms_ref[...] = ms.reshape(_SPAN_BLK)


_SPAN_BLK = 1024
_span_call = pl.pallas_call(
    _span_body,
    grid=((B * S) // _SPAN_BLK,),
    in_specs=[
        pl.BlockSpec((_SPAN_BLK, E), lambda i: (i, 0)),
        pl.BlockSpec((_SPAN_BLK, E), lambda i: (i, 0)),
        pl.BlockSpec((_SPAN_BLK, L), lambda i: (i, 0)),
        pl.BlockSpec((3 * E, H), lambda i: (0, 0)),
        pl.BlockSpec((1, H), lambda i: (0, 0)),
        pl.BlockSpec((H, H), lambda i: (0, 0)),
        pl.BlockSpec((1, H), lambda i: (0, 0)),
        pl.BlockSpec((H, 1), lambda i: (0, 0)),
        pl.BlockSpec((1, 1), lambda i: (0, 0)),
    ],
    out_specs=[
        pl.BlockSpec((_SPAN_BLK, 3 * E), lambda i: (i, 0)),
        pl.BlockSpec((_SPAN_BLK,), lambda i: (i,)),
    ],
    out_shape=[
        jax.ShapeDtypeStruct((B * S, 3 * E), jnp.float32),
        jax.ShapeDtypeStruct((B * S,), jnp.float32),
    ],
)


def kernel(batch_embeds, batch_spans_ids, aW1, ab1, aW2, ab2, aW3, ab3,
           sW1, sb1, sW2, sb2, sW3, sb3):
    ef = batch_embeds.reshape(B * T, E)
    idsf = batch_spans_ids.reshape(B * S * L).astype(jnp.int32)
    w = _token_call(ef.T, aW1.T, ab1, aW2, ab2,
                    aW3.reshape(1, H), ab3.reshape(1, 1))
    first, last, wgt = _sc_call(ef, idsf, w)
    se, ms = _span_call(first, last, wgt.reshape(B * S, L),
                        sW1, sb1.reshape(1, H), sW2, sb2.reshape(1, H),
                        sW3, sb3.reshape(1, 1))
    return se.reshape(B, S, 3 * E), ms.reshape(B, S, 1)


# speedup vs baseline: 1.0078x; 1.0078x over previous
import functools

import jax
import jax.numpy as jnp
from jax import lax
from jax.experimental import pallas as pl
from jax.experimental.pallas import tpu as pltpu
from jax.experimental.pallas import tpu_sc as plsc

B, T, E, S, L, H = 8, 2048, 128, 512, 128, 150
NC, NS = 2, 16
NW = NC * NS
SPW = (B * S) // NW
IDW = SPW * L
HI = lax.Precision.DEFAULT


def _token_body(xt_ref, w1t, b1, w2, b2, w3t, b3, out_ref):
    xt = xt_ref[...]
    b1c = b1[...].reshape(H, 1)
    b2c = b2[...].reshape(H, 1)
    ht = jnp.maximum(jnp.dot(w1t[...], xt, precision=HI) + b1c, 0.0)
    ht = jnp.maximum(
        lax.dot_general(w2[...], ht, (((0,), (0,)), ((), ())),
                        precision=HI) + b2c, 0.0)
    sct = jnp.dot(w3t[...], ht, precision=HI) + b3[...]
    rst = jnp.sum(xt, axis=0, keepdims=True)
    out_ref[...] = (sct * rst).reshape(_TOKEN_BLK)


_TOKEN_BLK = 4096
_token_call = pl.pallas_call(
    _token_body,
    grid=((B * T) // _TOKEN_BLK,),
    in_specs=[
        pl.BlockSpec((E, _TOKEN_BLK), lambda i: (0, i)),
        pl.BlockSpec((H, E), lambda i: (0, 0)),
        pl.BlockSpec((H,), lambda i: (0,)),
        pl.BlockSpec((H, H), lambda i: (0, 0)),
        pl.BlockSpec((H,), lambda i: (0,)),
        pl.BlockSpec((1, H), lambda i: (0, 0)),
        pl.BlockSpec((1, 1), lambda i: (0, 0)),
    ],
    out_specs=pl.BlockSpec((_TOKEN_BLK,), lambda i: (i,)),
    out_shape=jax.ShapeDtypeStruct((B * T,), jnp.float32),
)


def _sc_body(embeds, ids, w, first_o, last_o, wgt_o,
             ids_v, w_v, fidx, lidx, rows_f, rows_l, out_w, sem):
    wid = lax.axis_index("s") * NC + lax.axis_index("c")
    b = wid // (NW // B)
    span0 = wid * SPW
    bT = b * T

    pltpu.sync_copy(ids.at[pl.ds(wid * IDW, IDW)], ids_v)

    iota = lax.iota(jnp.int32, 16)
    for j in range(SPW // 16):
        pos = (j * 16 + iota) * L
        fidx[pl.ds(j * 16, 16)] = plsc.load_gather(ids_v, [pos]) + bT
        lidx[pl.ds(j * 16, 16)] = plsc.load_gather(ids_v, [pos + (L - 1)]) + bT

    cf = pltpu.async_copy(embeds.at[fidx], rows_f, sem)
    cl = pltpu.async_copy(embeds.at[lidx], rows_l, sem)

    pltpu.sync_copy(w.at[pl.ds(bT, T)], w_v)

    @plsc.parallel_loop(0, IDW, 16, unroll=8)
    def _(off):
        idx = ids_v[pl.ds(off, 16)]
        out_w[pl.ds(off, 16)] = plsc.load_gather(w_v, [idx])

    cf.wait()
    cl.wait()
    pltpu.sync_copy(rows_f, first_o.at[pl.ds(span0, SPW)])
    pltpu.sync_copy(rows_l, last_o.at[pl.ds(span0, SPW)])
    pltpu.sync_copy(out_w, wgt_o.at[pl.ds(wid * IDW, IDW)])


_sc_call = pl.kernel(
    _sc_body,
    out_type=[
        jax.ShapeDtypeStruct((B * S, E), jnp.float32),
        jax.ShapeDtypeStruct((B * S, E), jnp.float32),
        jax.ShapeDtypeStruct((B * S * L,), jnp.float32),
    ],
    mesh=plsc.VectorSubcoreMesh(
        core_axis_name="c", subcore_axis_name="s",
        num_cores=NC, num_subcores=NS),
    compiler_params=pltpu.CompilerParams(needs_layout_passes=False),
    scratch_types=[
        pltpu.VMEM((IDW,), jnp.int32),
        pltpu.VMEM((T,), jnp.float32),
        pltpu.VMEM((SPW,), jnp.int32),
        pltpu.VMEM((SPW,), jnp.int32),
        pltpu.VMEM((SPW, E), jnp.float32),
        pltpu.VMEM((SPW, E), jnp.float32),
        pltpu.VMEM((IDW,), jnp.float32),
        pltpu.SemaphoreType.DMA,
    ],
)


def _span_body(f_ref, l_ref, g_ref, w1, b1, w2, b2, w3, b3, se_ref, ms_ref):
    f = f_ref[...]
    lst = l_ref[...]
    g = g_ref[...]
    se_ref[:, 0:E] = f
    se_ref[:, E:2 * E] = lst
    se_ref[:, 2 * E:3 * E] = g
    h = (jnp.dot(f, w1[0:E, :], precision=HI)
         + jnp.dot(lst, w1[E:2 * E, :], precision=HI)
         + jnp.dot(g, w1[2 * E:3 * E, :], precision=HI) + b1[...])
    h = jnp.maximum(h, 0.0)
    h = jnp.maximum(jnp.dot(h, w2[...], precision=HI) + b2[...], 0.0)
    ms = jnp.dot(h, w3[...], precision=HI) + b3[...]
    ms_ref[...] = ms.reshape(_SPAN_BLK)


_SPAN_BLK = 1024
_span_call = pl.pallas_call(
    _span_body,
    grid=((B * S) // _SPAN_BLK,),
    in_specs=[
        pl.BlockSpec((_SPAN_BLK, E), lambda i: (i, 0)),
        pl.BlockSpec((_SPAN_BLK, E), lambda i: (i, 0)),
        pl.BlockSpec((_SPAN_BLK, L), lambda i: (i, 0)),
        pl.BlockSpec((3 * E, H), lambda i: (0, 0)),
        pl.BlockSpec((1, H), lambda i: (0, 0)),
        pl.BlockSpec((H, H), lambda i: (0, 0)),
        pl.BlockSpec((1, H), lambda i: (0, 0)),
        pl.BlockSpec((H, 1), lambda i: (0, 0)),
        pl.BlockSpec((1, 1), lambda i: (0, 0)),
    ],
    out_specs=[
        pl.BlockSpec((_SPAN_BLK, 3 * E), lambda i: (i, 0)),
        pl.BlockSpec((_SPAN_BLK,), lambda i: (i,)),
    ],
    out_shape=[
        jax.ShapeDtypeStruct((B * S, 3 * E), jnp.float32),
        jax.ShapeDtypeStruct((B * S,), jnp.float32),
    ],
)


def kernel(batch_embeds, batch_spans_ids, aW1, ab1, aW2, ab2, aW3, ab3,
           sW1, sb1, sW2, sb2, sW3, sb3):
    ef = batch_embeds.reshape(B * T, E)
    idsf = batch_spans_ids.reshape(B * S * L).astype(jnp.int32)
    w = _token_call(ef.T, aW1.T, ab1, aW2, ab2,
                    aW3.reshape(1, H), ab3.reshape(1, 1))
    first, last, wgt = _sc_call(ef, idsf, w)
    se, ms = _span_call(first, last, wgt.reshape(B * S, L),
                        sW1, sb1.reshape(1, H), sW2, sb2.reshape(1, H),
                        sW3, sb3.reshape(1, 1))
    return se.reshape(B, S, 3 * E), ms.reshape(B, S, 1)


# speedup vs baseline: 1.2035x; 1.1942x over previous
import functools

import jax
import jax.numpy as jnp
from jax import lax
from jax.experimental import pallas as pl
from jax.experimental.pallas import tpu as pltpu
from jax.experimental.pallas import tpu_sc as plsc

B, T, E, S, L, H = 8, 2048, 128, 512, 128, 150
NC, NS = 2, 16
NW = NC * NS
SPW = (B * S) // NW
IDW = SPW * L
HI = lax.Precision.DEFAULT


def _token_body(xt_ref, w1t, b1, w2, b2, w3t, b3, out_ref):
    xt = xt_ref[...].T
    b1c = b1[...].reshape(H, 1)
    b2c = b2[...].reshape(H, 1)
    ht = jnp.maximum(jnp.dot(w1t[...], xt, precision=HI) + b1c, 0.0)
    ht = jnp.maximum(
        lax.dot_general(w2[...], ht, (((0,), (0,)), ((), ())),
                        precision=HI) + b2c, 0.0)
    sct = jnp.dot(w3t[...], ht, precision=HI) + b3[...]
    rst = jnp.sum(xt, axis=0, keepdims=True)
    out_ref[...] = (sct * rst).reshape(_TOKEN_BLK)


_TOKEN_BLK = 4096
_token_call = pl.pallas_call(
    _token_body,
    grid=((B * T) // _TOKEN_BLK,),
    in_specs=[
        pl.BlockSpec((_TOKEN_BLK, E), lambda i: (i, 0)),
        pl.BlockSpec((H, E), lambda i: (0, 0)),
        pl.BlockSpec((H,), lambda i: (0,)),
        pl.BlockSpec((H, H), lambda i: (0, 0)),
        pl.BlockSpec((H,), lambda i: (0,)),
        pl.BlockSpec((1, H), lambda i: (0, 0)),
        pl.BlockSpec((1, 1), lambda i: (0, 0)),
    ],
    out_specs=pl.BlockSpec((_TOKEN_BLK,), lambda i: (i,)),
    out_shape=jax.ShapeDtypeStruct((B * T,), jnp.float32),
)


def _sc_body(embeds, ids, w, first_o, last_o, wgt_o,
             ids_v, w_v, fidx, lidx, rows_f, rows_l, out_w, sem):
    wid = lax.axis_index("s") * NC + lax.axis_index("c")
    b = wid // (NW // B)
    span0 = wid * SPW
    bT = b * T

    pltpu.sync_copy(ids.at[pl.ds(wid * IDW, IDW)], ids_v)

    iota = lax.iota(jnp.int32, 16)
    for j in range(SPW // 16):
        pos = (j * 16 + iota) * L
        fidx[pl.ds(j * 16, 16)] = plsc.load_gather(ids_v, [pos]) + bT
        lidx[pl.ds(j * 16, 16)] = plsc.load_gather(ids_v, [pos + (L - 1)]) + bT

    cf = pltpu.async_copy(embeds.at[fidx], rows_f, sem)
    cl = pltpu.async_copy(embeds.at[lidx], rows_l, sem)

    pltpu.sync_copy(w.at[pl.ds(bT, T)], w_v)

    def body(k, carry):
        for u in range(8):
            off = k * 128 + u * 16
            idx = ids_v[pl.ds(off, 16)]
            out_w[pl.ds(off, 16)] = plsc.load_gather(w_v, [idx])
        return carry

    lax.fori_loop(0, IDW // 128, body, 0)

    cf.wait()
    cl.wait()
    pltpu.sync_copy(rows_f, first_o.at[pl.ds(span0, SPW)])
    pltpu.sync_copy(rows_l, last_o.at[pl.ds(span0, SPW)])
    pltpu.sync_copy(out_w, wgt_o.at[pl.ds(wid * IDW, IDW)])


_sc_call = pl.kernel(
    _sc_body,
    out_type=[
        jax.ShapeDtypeStruct((B * S, E), jnp.float32),
        jax.ShapeDtypeStruct((B * S, E), jnp.float32),
        jax.ShapeDtypeStruct((B * S * L,), jnp.float32),
    ],
    mesh=plsc.VectorSubcoreMesh(
        core_axis_name="c", subcore_axis_name="s",
        num_cores=NC, num_subcores=NS),
    compiler_params=pltpu.CompilerParams(needs_layout_passes=False),
    scratch_types=[
        pltpu.VMEM((IDW,), jnp.int32),
        pltpu.VMEM((T,), jnp.float32),
        pltpu.VMEM((SPW,), jnp.int32),
        pltpu.VMEM((SPW,), jnp.int32),
        pltpu.VMEM((SPW, E), jnp.float32),
        pltpu.VMEM((SPW, E), jnp.float32),
        pltpu.VMEM((IDW,), jnp.float32),
        pltpu.SemaphoreType.DMA,
    ],
)


def _span_body(f_ref, l_ref, g_ref, w1, b1, w2, b2, w3, b3, se_ref, ms_ref):
    f = f_ref[...]
    lst = l_ref[...]
    g = g_ref[...]
    se_ref[:, 0:E] = f
    se_ref[:, E:2 * E] = lst
    se_ref[:, 2 * E:3 * E] = g
    h = (jnp.dot(f, w1[0:E, :], precision=HI)
         + jnp.dot(lst, w1[E:2 * E, :], precision=HI)
         + jnp.dot(g, w1[2 * E:3 * E, :], precision=HI) + b1[...])
    h = jnp.maximum(h, 0.0)
    h = jnp.maximum(jnp.dot(h, w2[...], precision=HI) + b2[...], 0.0)
    ms = jnp.dot(h, w3[...], precision=HI) + b3[...]
    ms_ref[...] = ms.reshape(_SPAN_BLK)


_SPAN_BLK = 1024
_span_call = pl.pallas_call(
    _span_body,
    grid=((B * S) // _SPAN_BLK,),
    in_specs=[
        pl.BlockSpec((_SPAN_BLK, E), lambda i: (i, 0)),
        pl.BlockSpec((_SPAN_BLK, E), lambda i: (i, 0)),
        pl.BlockSpec((_SPAN_BLK, L), lambda i: (i, 0)),
        pl.BlockSpec((3 * E, H), lambda i: (0, 0)),
        pl.BlockSpec((1, H), lambda i: (0, 0)),
        pl.BlockSpec((H, H), lambda i: (0, 0)),
        pl.BlockSpec((1, H), lambda i: (0, 0)),
        pl.BlockSpec((H, 1), lambda i: (0, 0)),
        pl.BlockSpec((1, 1), lambda i: (0, 0)),
    ],
    out_specs=[
        pl.BlockSpec((_SPAN_BLK, 3 * E), lambda i: (i, 0)),
        pl.BlockSpec((_SPAN_BLK,), lambda i: (i,)),
    ],
    out_shape=[
        jax.ShapeDtypeStruct((B * S, 3 * E), jnp.float32),
        jax.ShapeDtypeStruct((B * S,), jnp.float32),
    ],
)


def kernel(batch_embeds, batch_spans_ids, aW1, ab1, aW2, ab2, aW3, ab3,
           sW1, sb1, sW2, sb2, sW3, sb3):
    ef = batch_embeds.reshape(B * T, E)
    idsf = batch_spans_ids.reshape(B * S * L).astype(jnp.int32)
    w = _token_call(ef, aW1.T, ab1, aW2, ab2,
                    aW3.reshape(1, H), ab3.reshape(1, 1))
    first, last, wgt = _sc_call(ef, idsf, w)
    se, ms = _span_call(first, last, wgt.reshape(B * S, L),
                        sW1, sb1.reshape(1, H), sW2, sb2.reshape(1, H),
                        sW3, sb3.reshape(1, 1))
    return se.reshape(B, S, 3 * E), ms.reshape(B, S, 1)
